# Initial kernel scaffold; baseline (speedup 1.0000x reference)
import jax
import jax.numpy as jnp
from jax.experimental import pallas as pl


def kernel(input_features, qw_re, qw_im, rotation_angles, entanglement_matrix, meas_W, meas_b):
    raise NotImplementedError("write your pallas kernel here")


# speedup vs baseline: 1.3027x; 1.3027x over previous
import numpy as np
import jax
import jax.numpy as jnp
from jax import lax
from jax.experimental import pallas as pl
from jax.experimental.pallas import tpu as pltpu

NQ = 16
S = 1 << 16
D = 128
B = 4
ENT = 0.8
NQS = S // NQ
SD = S * D
KC = 8
NCORE = 1
TSTEPS = NQS // KC // NCORE

_ROTS = ((13, 15, 26, 6), (17, 29, 16, 24))


def _np_threefry2x32(k0, k1, x0, x1):
    k0 = np.uint32(k0); k1 = np.uint32(k1)
    ks = [k0, k1, np.uint32(k0 ^ k1 ^ np.uint32(0x1BD11BDA))]
    x0 = (x0 + k0).astype(np.uint32)
    x1 = (x1 + k1).astype(np.uint32)
    for i in range(5):
        for r in _ROTS[i % 2]:
            x0 = (x0 + x1).astype(np.uint32)
            x1 = ((x1 << np.uint32(r)) | (x1 >> np.uint32(32 - r))) ^ x0
        x0 = (x0 + ks[(i + 1) % 3]).astype(np.uint32)
        x1 = (x1 + ks[(i + 2) % 3] + np.uint32(i + 1)).astype(np.uint32)
    return x0, x1


_Y0, _Y1 = _np_threefry2x32(0, 1, np.zeros(3, np.uint32), np.arange(3, dtype=np.uint32))
_KEYS = tuple((int(_Y0[j]), int(_Y1[j])) for j in range(3))


def _tf_bits(key, lo):
    k0, k1 = key
    ks = (np.uint32(k0), np.uint32(k1),
          np.uint32(k0 ^ k1 ^ 0x1BD11BDA))
    x0 = jnp.full_like(lo, np.uint32(k0))
    x1 = lo + np.uint32(k1)
    for i in range(5):
        for r in _ROTS[i % 2]:
            x0 = x0 + x1
            x1 = ((x1 << np.uint32(r)) | (x1 >> np.uint32(32 - r))) ^ x0
        x0 = x0 + ks[(i + 1) % 3]
        x1 = x1 + np.uint32((int(ks[(i + 2) % 3]) + i + 1) & 0xFFFFFFFF)
    return x0 ^ x1


def _bits_to_unit(bits):
    f = lax.bitcast_convert_type((bits >> np.uint32(9)) | np.uint32(0x3F800000),
                                 jnp.float32)
    return f - np.float32(1.0)


_ERFINV_SMALL = (2.81022636e-08, 3.43273939e-07, -3.5233877e-06, -4.39150654e-06,
                 0.00021858087, -0.00125372503, -0.00417768164, 0.246640727,
                 1.50140941)
_ERFINV_BIG = (-0.000200214257, 0.000100950558, 0.00134934322, -0.00367342844,
               0.00573950773, -0.0076224613, 0.00943887047, 1.00167406,
               2.83297682)


def _erfinv(x):
    w = -jnp.log1p(-x * x)
    ws = w - np.float32(2.5)
    wb = jnp.sqrt(w) - np.float32(3.0)
    ps = np.float32(_ERFINV_SMALL[0])
    for c in _ERFINV_SMALL[1:]:
        ps = ps * ws + np.float32(c)
    pb = np.float32(_ERFINV_BIG[0])
    for c in _ERFINV_BIG[1:]:
        pb = pb * wb + np.float32(c)
    return jnp.where(w < np.float32(5.0), ps, pb) * x


_ULO = np.float32(np.nextafter(np.float32(-1.0), np.float32(0.0)))
_UHI = np.float32(1.0)
_SQRT2 = np.float32(np.sqrt(2.0))
_UEPS = np.float32(1e-7)


def _main_body(qre_ref, qim_ref, ew_ref, par_ref, bv_ref, bi_ref):
    c = pl.program_id(0)
    t = pl.program_id(1)

    @pl.when(t == 0)
    def _():
        bv_ref[...] = jnp.full((1, B, D), -np.inf, jnp.float32)
        bi_ref[...] = jnp.zeros((1, B, D), jnp.int32)

    qre = qre_ref[...]
    qim = qim_ref[...]
    ew = ew_ref[...]
    dn = (((1,), (0,)), ((), ()))
    a_re = lax.dot_general(ew, qre, dn, preferred_element_type=jnp.float32)
    a_im = lax.dot_general(ew, qim, dn, preferred_element_type=jnp.float32)
    cc = par_ref[0]
    sc = par_ref[1]
    ns = par_ref[2]
    xr = cc * a_re - sc * a_im
    xi = sc * a_re + cc * a_im

    base_k = ((c * TSTEPS + t) * KC).astype(jnp.uint32)
    q_io = lax.broadcasted_iota(jnp.uint32, (NQ, KC, D), 0)
    k_io = lax.broadcasted_iota(jnp.uint32, (NQ, KC, D), 1)
    d_io = lax.broadcasted_iota(jnp.uint32, (NQ, KC, D), 2)
    s_idx = q_io * np.uint32(NQS) + (base_k + k_io)
    i0 = s_idx * np.uint32(D) + d_io
    s_i32 = s_idx.astype(jnp.int32)

    for b in range(B):
        lo = i0 + np.uint32(b * SD)
        u = jnp.maximum(_UEPS,
                        _bits_to_unit(_tf_bits(_KEYS[2], lo))
                        * (np.float32(1.0) - _UEPS) + _UEPS)
        g = jnp.log(-jnp.log(u))
        u0 = jnp.maximum(_ULO, _bits_to_unit(_tf_bits(_KEYS[0], lo))
                         * (_UHI - _ULO) + _ULO)
        u1 = jnp.maximum(_ULO, _bits_to_unit(_tf_bits(_KEYS[1], lo))
                         * (_UHI - _ULO) + _ULO)
        x = xr + ns * _erfinv(u0)
        y = xi + ns * _erfinv(u1)
        score = jnp.log(x * x + y * y + np.float32(1e-20)) - g
        m = jnp.max(jnp.max(score, axis=0), axis=0)[None]
        cand = jnp.min(jnp.min(jnp.where(score == m, s_i32, jnp.int32(0x7FFFFFFF)),
                               axis=0), axis=0)[None]
        old_v = bv_ref[0, b][None]
        old_i = bi_ref[0, b][None]
        better = (m > old_v) | ((m == old_v) & (cand < old_i))
        bv_ref[0, b] = jnp.where(better, m, old_v)[0]
        bi_ref[0, b] = jnp.where(better, cand, old_i)[0]


def _final_body(bv_ref, bi_ref, w_ref, mb_ref, o_ref):
    v0 = bv_ref[0]
    v1 = bv_ref[1]
    i0_ = bi_ref[0]
    i1_ = bi_ref[1]
    take1 = (v1 > v0) | ((v1 == v0) & (i1_ < i0_))
    idx = jnp.where(take1, i1_, i0_)
    acc = jnp.broadcast_to(jnp.sum(mb_ref[...], axis=0)[None], (B, D))
    dn = (((1,), (1,)), ((), ()))
    for n in range(NQ):
        mask = (idx == n).astype(jnp.float32)
        acc = acc + lax.dot_general(mask, w_ref[n], dn,
                                    preferred_element_type=jnp.float32)
    o_ref[...] = acc * np.float32(1.0 / NQ)


def kernel(input_features, qw_re, qw_im, rotation_angles, entanglement_matrix,
           meas_W, meas_b):
    del input_features
    coh = np.float32(np.exp(np.float32(-1.0 / 100.0)))
    theta = jnp.sum(rotation_angles)
    cc = coh * jnp.cos(theta)
    sc = coh * jnp.sin(theta)
    nscale = np.float32(1.0 / np.sqrt(2.0)) * (np.float32(1.0) - coh) * np.float32(0.1)
    params = jnp.stack([cc, sc, nscale * _SQRT2, jnp.float32(0.0)])
    ew = jax.nn.softmax(entanglement_matrix, axis=-1)
    ew_eff = np.float32(ENT) * ew + np.float32(1.0 - ENT) * jnp.eye(NQ, dtype=jnp.float32)

    qw3_re = qw_re.reshape(NQ, NQS, D)
    qw3_im = qw_im.reshape(NQ, NQS, D)

    bv, bi = pl.pallas_call(
        _main_body,
        grid=(NCORE, TSTEPS),
        in_specs=[
            pl.BlockSpec((NQ, KC, D), lambda c, t: (0, c * TSTEPS + t, 0)),
            pl.BlockSpec((NQ, KC, D), lambda c, t: (0, c * TSTEPS + t, 0)),
            pl.BlockSpec((NQ, NQ), lambda c, t: (0, 0)),
            pl.BlockSpec(memory_space=pltpu.SMEM),
        ],
        out_specs=[
            pl.BlockSpec((1, B, D), lambda c, t: (c, 0, 0)),
            pl.BlockSpec((1, B, D), lambda c, t: (c, 0, 0)),
        ],
        out_shape=[
            jax.ShapeDtypeStruct((NCORE, B, D), jnp.float32),
            jax.ShapeDtypeStruct((NCORE, B, D), jnp.int32),
        ],
        compiler_params=pltpu.CompilerParams(
            dimension_semantics=("arbitrary", "arbitrary"),
        ),
    )(qw3_re, qw3_im, ew_eff, params)

    out = pl.pallas_call(
        _final_body,
        out_shape=jax.ShapeDtypeStruct((B, D), jnp.float32),
    )(bv, bi, meas_W, meas_b)
    return out


# speedup vs baseline: 4.1811x; 3.2095x over previous
import numpy as np
import jax
import jax.numpy as jnp
from jax import lax
from jax.experimental import pallas as pl
from jax.experimental.pallas import tpu as pltpu

NQ = 16
S = 1 << 16
D = 128
B = 4
ENT = 0.8
NQS = S // NQ
SD = S * D
KC = 8
TSTEPS = NQS // KC
TOPK = 3

_ROTS = ((13, 15, 26, 6), (17, 29, 16, 24))


def _np_threefry2x32(k0, k1, x0, x1):
    k0 = np.uint32(k0); k1 = np.uint32(k1)
    ks = [k0, k1, np.uint32(k0 ^ k1 ^ np.uint32(0x1BD11BDA))]
    x0 = (x0 + k0).astype(np.uint32)
    x1 = (x1 + k1).astype(np.uint32)
    for i in range(5):
        for r in _ROTS[i % 2]:
            x0 = (x0 + x1).astype(np.uint32)
            x1 = ((x1 << np.uint32(r)) | (x1 >> np.uint32(32 - r))) ^ x0
        x0 = (x0 + ks[(i + 1) % 3]).astype(np.uint32)
        x1 = (x1 + ks[(i + 2) % 3] + np.uint32(i + 1)).astype(np.uint32)
    return x0, x1


_Y0, _Y1 = _np_threefry2x32(0, 1, np.zeros(3, np.uint32), np.arange(3, dtype=np.uint32))
_KEYS = tuple((int(_Y0[j]), int(_Y1[j])) for j in range(3))


def _tf_bits(key, lo):
    k0, k1 = key
    ks = (np.uint32(k0), np.uint32(k1),
          np.uint32(k0 ^ k1 ^ 0x1BD11BDA))
    x0 = jnp.full_like(lo, np.uint32(k0))
    x1 = lo + np.uint32(k1)
    for i in range(5):
        for r in _ROTS[i % 2]:
            x0 = x0 + x1
            x1 = ((x1 << np.uint32(r)) | (x1 >> np.uint32(32 - r))) ^ x0
        x0 = x0 + ks[(i + 1) % 3]
        x1 = x1 + np.uint32((int(ks[(i + 2) % 3]) + i + 1) & 0xFFFFFFFF)
    return x0 ^ x1


def _bits_to_unit(bits):
    f = lax.bitcast_convert_type((bits >> np.uint32(9)) | np.uint32(0x3F800000),
                                 jnp.float32)
    return f - np.float32(1.0)


_ERFINV_SMALL = (2.81022636e-08, 3.43273939e-07, -3.5233877e-06, -4.39150654e-06,
                 0.00021858087, -0.00125372503, -0.00417768164, 0.246640727,
                 1.50140941)
_ERFINV_BIG = (-0.000200214257, 0.000100950558, 0.00134934322, -0.00367342844,
               0.00573950773, -0.0076224613, 0.00943887047, 1.00167406,
               2.83297682)


def _erfinv(x):
    w = -jnp.log1p(-x * x)
    ws = w - np.float32(2.5)
    wb = jnp.sqrt(w) - np.float32(3.0)
    ps = np.float32(_ERFINV_SMALL[0])
    for c in _ERFINV_SMALL[1:]:
        ps = ps * ws + np.float32(c)
    pb = np.float32(_ERFINV_BIG[0])
    for c in _ERFINV_BIG[1:]:
        pb = pb * wb + np.float32(c)
    return jnp.where(w < np.float32(5.0), ps, pb) * x


_ULO = np.float32(np.nextafter(np.float32(-1.0), np.float32(0.0)))
_UHI = np.float32(1.0)
_SQRT2 = np.float32(np.sqrt(2.0))
_UEPS = np.float32(1e-7)
_MAXI = np.int32(0x7FFFFFFF)


def _gumbel_u(lo):
    return jnp.maximum(_UEPS, _bits_to_unit(_tf_bits(_KEYS[2], lo))
                       * (np.float32(1.0) - _UEPS) + _UEPS)


def _noise_normal(key, lo):
    u = jnp.maximum(_ULO, _bits_to_unit(_tf_bits(key, lo)) * (_UHI - _ULO) + _ULO)
    return _erfinv(u)


def _main_body(qre_ref, qim_ref, ew_ref, par_ref, cv_ref, cs_ref, cxr_ref, cxi_ref):
    t = pl.program_id(0)

    @pl.when(t == 0)
    def _():
        cv_ref[...] = jnp.full((TOPK, B, D), -np.inf, jnp.float32)
        cs_ref[...] = jnp.full((TOPK, B, D), _MAXI, jnp.int32)
        cxr_ref[...] = jnp.zeros((TOPK, B, D), jnp.float32)
        cxi_ref[...] = jnp.zeros((TOPK, B, D), jnp.float32)

    qre = qre_ref[...]
    qim = qim_ref[...]
    ew = ew_ref[...]
    dn = (((1,), (0,)), ((), ()))
    a_re = lax.dot_general(ew, qre, dn, preferred_element_type=jnp.float32)
    a_im = lax.dot_general(ew, qim, dn, preferred_element_type=jnp.float32)
    cc = par_ref[0]
    sc = par_ref[1]
    xr = cc * a_re - sc * a_im
    xi = sc * a_re + cc * a_im
    logits = jnp.log(xr * xr + xi * xi + np.float32(1e-20))

    base_k = (t * KC).astype(jnp.uint32)
    q_io = lax.broadcasted_iota(jnp.uint32, (NQ, KC, D), 0)
    k_io = lax.broadcasted_iota(jnp.uint32, (NQ, KC, D), 1)
    d_io = lax.broadcasted_iota(jnp.uint32, (NQ, KC, D), 2)
    s_idx = q_io * np.uint32(NQS) + (base_k + k_io)
    i0 = s_idx * np.uint32(D) + d_io
    s_i32 = s_idx.astype(jnp.int32)

    for b in range(B):
        u = _gumbel_u(i0 + np.uint32(b * SD))
        score = logits - jnp.log(-jnp.log(u))
        m = jnp.max(jnp.max(score, axis=0), axis=0)[None]
        cand = jnp.min(jnp.min(jnp.where(score == m, s_i32, _MAXI),
                               axis=0), axis=0)[None]
        sel = s_i32 == cand
        wxr = jnp.max(jnp.max(jnp.where(sel, xr, -np.inf), axis=0), axis=0)[None]
        wxi = jnp.max(jnp.max(jnp.where(sel, xi, -np.inf), axis=0), axis=0)[None]
        nv, ns_, nxr, nxi = m, cand, wxr, wxi
        for k in range(TOPK):
            ov = cv_ref[k, b][None]
            os_ = cs_ref[k, b][None]
            oxr = cxr_ref[k, b][None]
            oxi = cxi_ref[k, b][None]
            better = (nv > ov) | ((nv == ov) & (ns_ < os_))
            cv_ref[k, b] = jnp.where(better, nv, ov)[0]
            cs_ref[k, b] = jnp.where(better, ns_, os_)[0]
            cxr_ref[k, b] = jnp.where(better, nxr, oxr)[0]
            cxi_ref[k, b] = jnp.where(better, nxi, oxi)[0]
            nv = jnp.where(better, ov, nv)
            ns_ = jnp.where(better, os_, ns_)
            nxr = jnp.where(better, oxr, nxr)
            nxi = jnp.where(better, oxi, nxi)


def _final_body(cs_ref, cxr_ref, cxi_ref, par_ref, w_ref, mb_ref, o_ref):
    s_c = cs_ref[...]
    xr = cxr_ref[...]
    xi = cxi_ref[...]
    ns = par_ref[2]
    b_io = lax.broadcasted_iota(jnp.uint32, (TOPK, B, D), 1) * np.uint32(SD)
    d_io = lax.broadcasted_iota(jnp.uint32, (TOPK, B, D), 2)
    lo = s_c.astype(jnp.uint32) * np.uint32(D) + b_io + d_io
    x = xr + ns * _noise_normal(_KEYS[0], lo)
    y = xi + ns * _noise_normal(_KEYS[1], lo)
    u = _gumbel_u(lo)
    score = jnp.log(x * x + y * y + np.float32(1e-20)) - jnp.log(-jnp.log(u))
    bv = score[0]
    bi = s_c[0]
    for k in range(1, TOPK):
        kv = score[k]
        ki = s_c[k]
        better = (kv > bv) | ((kv == bv) & (ki < bi))
        bv = jnp.where(better, kv, bv)
        bi = jnp.where(better, ki, bi)
    acc = jnp.broadcast_to(jnp.sum(mb_ref[...], axis=0)[None], (B, D))
    dn = (((1,), (1,)), ((), ()))
    for n in range(NQ):
        mask = (bi == n).astype(jnp.float32)
        acc = acc + lax.dot_general(mask, w_ref[n], dn,
                                    preferred_element_type=jnp.float32)
    o_ref[...] = acc * np.float32(1.0 / NQ)


def kernel(input_features, qw_re, qw_im, rotation_angles, entanglement_matrix,
           meas_W, meas_b):
    del input_features
    coh = np.float32(np.exp(np.float32(-1.0 / 100.0)))
    theta = jnp.sum(rotation_angles)
    cc = coh * jnp.cos(theta)
    sc = coh * jnp.sin(theta)
    nscale = np.float32(1.0 / np.sqrt(2.0)) * (np.float32(1.0) - coh) * np.float32(0.1)
    params = jnp.stack([cc, sc, nscale * _SQRT2, jnp.float32(0.0)])
    ew = jax.nn.softmax(entanglement_matrix, axis=-1)
    ew_eff = np.float32(ENT) * ew + np.float32(1.0 - ENT) * jnp.eye(NQ, dtype=jnp.float32)

    qw3_re = qw_re.reshape(NQ, NQS, D)
    qw3_im = qw_im.reshape(NQ, NQS, D)

    _, cs, cxr, cxi = pl.pallas_call(
        _main_body,
        grid=(TSTEPS,),
        in_specs=[
            pl.BlockSpec((NQ, KC, D), lambda t: (0, t, 0)),
            pl.BlockSpec((NQ, KC, D), lambda t: (0, t, 0)),
            pl.BlockSpec((NQ, NQ), lambda t: (0, 0)),
            pl.BlockSpec(memory_space=pltpu.SMEM),
        ],
        out_specs=[
            pl.BlockSpec((TOPK, B, D), lambda t: (0, 0, 0)),
            pl.BlockSpec((TOPK, B, D), lambda t: (0, 0, 0)),
            pl.BlockSpec((TOPK, B, D), lambda t: (0, 0, 0)),
            pl.BlockSpec((TOPK, B, D), lambda t: (0, 0, 0)),
        ],
        out_shape=[
            jax.ShapeDtypeStruct((TOPK, B, D), jnp.float32),
            jax.ShapeDtypeStruct((TOPK, B, D), jnp.int32),
            jax.ShapeDtypeStruct((TOPK, B, D), jnp.float32),
            jax.ShapeDtypeStruct((TOPK, B, D), jnp.float32),
        ],
        compiler_params=pltpu.CompilerParams(
            dimension_semantics=("arbitrary",),
        ),
    )(qw3_re, qw3_im, ew_eff, params)

    out = pl.pallas_call(
        _final_body,
        out_shape=jax.ShapeDtypeStruct((B, D), jnp.float32),
    )(cs, cxr, cxi, params, meas_W, meas_b)
    return out


# speedup vs baseline: 4.4168x; 1.0564x over previous
import numpy as np
import jax
import jax.numpy as jnp
from jax import lax
from jax.experimental import pallas as pl
from jax.experimental.pallas import tpu as pltpu

NQ = 16
S = 1 << 16
D = 128
B = 4
ENT = 0.8
NQS = S // NQ
SD = S * D
KC = 8
TSTEPS = NQS // KC
TOPK = 3

_ROTS = ((13, 15, 26, 6), (17, 29, 16, 24))


def _np_threefry2x32(k0, k1, x0, x1):
    k0 = np.uint32(k0); k1 = np.uint32(k1)
    ks = [k0, k1, np.uint32(k0 ^ k1 ^ np.uint32(0x1BD11BDA))]
    x0 = (x0 + k0).astype(np.uint32)
    x1 = (x1 + k1).astype(np.uint32)
    for i in range(5):
        for r in _ROTS[i % 2]:
            x0 = (x0 + x1).astype(np.uint32)
            x1 = ((x1 << np.uint32(r)) | (x1 >> np.uint32(32 - r))) ^ x0
        x0 = (x0 + ks[(i + 1) % 3]).astype(np.uint32)
        x1 = (x1 + ks[(i + 2) % 3] + np.uint32(i + 1)).astype(np.uint32)
    return x0, x1


_Y0, _Y1 = _np_threefry2x32(0, 1, np.zeros(3, np.uint32), np.arange(3, dtype=np.uint32))
_KEYS = tuple((int(_Y0[j]), int(_Y1[j])) for j in range(3))


def _tf_bits(key, lo):
    k0, k1 = key
    ks = (np.uint32(k0), np.uint32(k1),
          np.uint32(k0 ^ k1 ^ 0x1BD11BDA))
    x0 = jnp.full_like(lo, np.uint32(k0))
    x1 = lo + np.uint32(k1)
    for i in range(5):
        for r in _ROTS[i % 2]:
            x0 = x0 + x1
            x1 = ((x1 << np.uint32(r)) | (x1 >> np.uint32(32 - r))) ^ x0
        x0 = x0 + ks[(i + 1) % 3]
        x1 = x1 + np.uint32((int(ks[(i + 2) % 3]) + i + 1) & 0xFFFFFFFF)
    return x0 ^ x1


def _bits_to_unit(bits):
    f = lax.bitcast_convert_type((bits >> np.uint32(9)) | np.uint32(0x3F800000),
                                 jnp.float32)
    return f - np.float32(1.0)


_ERFINV_SMALL = (2.81022636e-08, 3.43273939e-07, -3.5233877e-06, -4.39150654e-06,
                 0.00021858087, -0.00125372503, -0.00417768164, 0.246640727,
                 1.50140941)
_ERFINV_BIG = (-0.000200214257, 0.000100950558, 0.00134934322, -0.00367342844,
               0.00573950773, -0.0076224613, 0.00943887047, 1.00167406,
               2.83297682)


def _erfinv(x):
    w = -jnp.log1p(-x * x)
    ws = w - np.float32(2.5)
    wb = jnp.sqrt(w) - np.float32(3.0)
    ps = np.float32(_ERFINV_SMALL[0])
    for c in _ERFINV_SMALL[1:]:
        ps = ps * ws + np.float32(c)
    pb = np.float32(_ERFINV_BIG[0])
    for c in _ERFINV_BIG[1:]:
        pb = pb * wb + np.float32(c)
    return jnp.where(w < np.float32(5.0), ps, pb) * x


_ULO = np.float32(np.nextafter(np.float32(-1.0), np.float32(0.0)))
_UHI = np.float32(1.0)
_SQRT2 = np.float32(np.sqrt(2.0))
_UEPS = np.float32(1e-7)
_MAXI = np.int32(0x7FFFFFFF)


def _gumbel_u(lo):
    return jnp.maximum(_UEPS, _bits_to_unit(_tf_bits(_KEYS[2], lo))
                       * (np.float32(1.0) - _UEPS) + _UEPS)


def _noise_normal(key, lo):
    u = jnp.maximum(_ULO, _bits_to_unit(_tf_bits(key, lo)) * (_UHI - _ULO) + _ULO)
    return _erfinv(u)


def _main_body(qre_ref, qim_ref, ew_ref, par_ref, cv_ref, cs_ref, cxr_ref,
               cxi_ref, lo_ref):
    t = pl.program_id(0)

    @pl.when(t == 0)
    def _():
        cv_ref[...] = jnp.full((TOPK, B, D), -np.float32(1.0), jnp.float32)
        cs_ref[...] = jnp.full((TOPK, B, D), _MAXI, jnp.int32)
        cxr_ref[...] = jnp.zeros((TOPK, B, D), jnp.float32)
        cxi_ref[...] = jnp.zeros((TOPK, B, D), jnp.float32)
        q_io = lax.broadcasted_iota(jnp.uint32, (NQ, KC, D), 0)
        k_io = lax.broadcasted_iota(jnp.uint32, (NQ, KC, D), 1)
        d_io = lax.broadcasted_iota(jnp.uint32, (NQ, KC, D), 2)
        lo_ref[...] = (q_io * np.uint32(NQS) + k_io) * np.uint32(D) + d_io

    qre = qre_ref[...]
    qim = qim_ref[...]
    ew = ew_ref[...]
    dn = (((1,), (0,)), ((), ()))
    a_re = lax.dot_general(ew, qre, dn, preferred_element_type=jnp.float32)
    a_im = lax.dot_general(ew, qim, dn, preferred_element_type=jnp.float32)
    cc = par_ref[0]
    sc = par_ref[1]
    xr = cc * a_re - sc * a_im
    xi = sc * a_re + cc * a_im
    ev = xr * xr + xi * xi + np.float32(1e-20)

    base = (t * (KC * D)).astype(jnp.uint32)
    lo_c = lo_ref[...]
    s_i32 = lax.shift_right_logical(lo_c + base, np.uint32(7)).astype(jnp.int32)

    ms, cands, wxrs, wxis = [], [], [], []
    for b in range(B):
        f = _bits_to_unit(_tf_bits(_KEYS[2], lo_c + (base + np.uint32(b * SD))))
        r = ev / (-jnp.log2(f))
        m = jnp.max(jnp.max(r, axis=0), axis=0)[None]
        cand = jnp.min(jnp.min(jnp.where(r == m, s_i32, _MAXI),
                               axis=0), axis=0)[None]
        sel = s_i32 == cand
        wxr = jnp.max(jnp.max(jnp.where(sel, xr, -np.inf), axis=0), axis=0)[None]
        wxi = jnp.max(jnp.max(jnp.where(sel, xi, -np.inf), axis=0), axis=0)[None]
        ms.append(m); cands.append(cand); wxrs.append(wxr); wxis.append(wxi)

    nv = jnp.concatenate(ms, axis=0)
    ns_ = jnp.concatenate(cands, axis=0)
    nxr = jnp.concatenate(wxrs, axis=0)
    nxi = jnp.concatenate(wxis, axis=0)
    for k in range(TOPK):
        ov = cv_ref[k]
        os_ = cs_ref[k]
        oxr = cxr_ref[k]
        oxi = cxi_ref[k]
        better = (nv > ov) | ((nv == ov) & (ns_ < os_))
        cv_ref[k] = jnp.where(better, nv, ov)
        cs_ref[k] = jnp.where(better, ns_, os_)
        cxr_ref[k] = jnp.where(better, nxr, oxr)
        cxi_ref[k] = jnp.where(better, nxi, oxi)
        nv = jnp.where(better, ov, nv)
        ns_ = jnp.where(better, os_, ns_)
        nxr = jnp.where(better, oxr, nxr)
        nxi = jnp.where(better, oxi, nxi)


def _final_body(cs_ref, cxr_ref, cxi_ref, par_ref, w_ref, mb_ref, o_ref):
    s_c = cs_ref[...]
    xr = cxr_ref[...]
    xi = cxi_ref[...]
    ns = par_ref[2]
    b_io = lax.broadcasted_iota(jnp.uint32, (TOPK, B, D), 1) * np.uint32(SD)
    d_io = lax.broadcasted_iota(jnp.uint32, (TOPK, B, D), 2)
    lo = s_c.astype(jnp.uint32) * np.uint32(D) + b_io + d_io
    x = xr + ns * _noise_normal(_KEYS[0], lo)
    y = xi + ns * _noise_normal(_KEYS[1], lo)
    u = _gumbel_u(lo)
    score = jnp.log(x * x + y * y + np.float32(1e-20)) - jnp.log(-jnp.log(u))
    bv = score[0]
    bi = s_c[0]
    for k in range(1, TOPK):
        kv = score[k]
        ki = s_c[k]
        better = (kv > bv) | ((kv == bv) & (ki < bi))
        bv = jnp.where(better, kv, bv)
        bi = jnp.where(better, ki, bi)
    acc = jnp.broadcast_to(jnp.sum(mb_ref[...], axis=0)[None], (B, D))
    dn = (((1,), (1,)), ((), ()))
    for n in range(NQ):
        mask = (bi == n).astype(jnp.float32)
        acc = acc + lax.dot_general(mask, w_ref[n], dn,
                                    preferred_element_type=jnp.float32)
    o_ref[...] = acc * np.float32(1.0 / NQ)


def kernel(input_features, qw_re, qw_im, rotation_angles, entanglement_matrix,
           meas_W, meas_b):
    del input_features
    coh = np.float32(np.exp(np.float32(-1.0 / 100.0)))
    theta = jnp.sum(rotation_angles)
    cc = coh * jnp.cos(theta)
    sc = coh * jnp.sin(theta)
    nscale = np.float32(1.0 / np.sqrt(2.0)) * (np.float32(1.0) - coh) * np.float32(0.1)
    params = jnp.stack([cc, sc, nscale * _SQRT2, jnp.float32(0.0)])
    ew = jax.nn.softmax(entanglement_matrix, axis=-1)
    ew_eff = np.float32(ENT) * ew + np.float32(1.0 - ENT) * jnp.eye(NQ, dtype=jnp.float32)

    qw3_re = qw_re.reshape(NQ, NQS, D)
    qw3_im = qw_im.reshape(NQ, NQS, D)

    _, cs, cxr, cxi = pl.pallas_call(
        _main_body,
        grid=(TSTEPS,),
        in_specs=[
            pl.BlockSpec((NQ, KC, D), lambda t: (0, t, 0)),
            pl.BlockSpec((NQ, KC, D), lambda t: (0, t, 0)),
            pl.BlockSpec((NQ, NQ), lambda t: (0, 0)),
            pl.BlockSpec(memory_space=pltpu.SMEM),
        ],
        out_specs=[
            pl.BlockSpec((TOPK, B, D), lambda t: (0, 0, 0)),
            pl.BlockSpec((TOPK, B, D), lambda t: (0, 0, 0)),
            pl.BlockSpec((TOPK, B, D), lambda t: (0, 0, 0)),
            pl.BlockSpec((TOPK, B, D), lambda t: (0, 0, 0)),
        ],
        out_shape=[
            jax.ShapeDtypeStruct((TOPK, B, D), jnp.float32),
            jax.ShapeDtypeStruct((TOPK, B, D), jnp.int32),
            jax.ShapeDtypeStruct((TOPK, B, D), jnp.float32),
            jax.ShapeDtypeStruct((TOPK, B, D), jnp.float32),
        ],
        scratch_shapes=[pltpu.VMEM((NQ, KC, D), jnp.uint32)],
        compiler_params=pltpu.CompilerParams(
            dimension_semantics=("arbitrary",),
        ),
    )(qw3_re, qw3_im, ew_eff, params)

    out = pl.pallas_call(
        _final_body,
        out_shape=jax.ShapeDtypeStruct((B, D), jnp.float32),
    )(cs, cxr, cxi, params, meas_W, meas_b)
    return out
